# baseline (speedup 1.0000x reference)
import functools

import jax
import jax.numpy as jnp
from jax import lax
from jax.experimental import pallas as pl
from jax.experimental.pallas import tpu as pltpu
from jax.experimental.pallas import tpu_sc as plsc

B = 4096
L = 200
D = 64
V = 100004
NC = 2
NS = 16
NW = NC * NS
RPW = B // NW
LP = 208
SPLIT = 128
CVT_ROWS = 3125
CVT_CHUNK = 125

_SC_PARAMS = pltpu.CompilerParams(
    use_tc_tiling_on_sc=False, needs_layout_passes=False)
_MESH = plsc.VectorSubcoreMesh(core_axis_name="c", subcore_axis_name="s")


@functools.partial(
    pl.kernel,
    out_type=jax.ShapeDtypeStruct((V, D), jnp.bfloat16),
    mesh=_MESH,
    compiler_params=_SC_PARAMS,
    scratch_types=[
        pltpu.VMEM((CVT_CHUNK, D), jnp.float32),
        pltpu.VMEM((CVT_CHUNK, D), jnp.bfloat16),
    ],
)
def _to_bf16(table_h, out_h, inb, outb):
    wid = lax.axis_index("s") * NC + lax.axis_index("c")
    start = wid * CVT_ROWS

    def convert_rows(n):
        def body(l, carry):
            for c in range(2):
                a = inb[l, pl.ds(32 * c, 16)]
                b = inb[l, pl.ds(32 * c + 16, 16)]
                outb[l, pl.ds(32 * c, 32)] = plsc.pack(
                    a, b, format=plsc.PackFormat.INTERLEAVED)
            return carry

        lax.fori_loop(0, n, body, 0, unroll=8)

    def chunk(g, carry):
        base = start + g * CVT_CHUNK
        pltpu.sync_copy(table_h.at[pl.ds(base, CVT_CHUNK)], inb)
        convert_rows(CVT_CHUNK)
        pltpu.sync_copy(outb, out_h.at[pl.ds(base, CVT_CHUNK)])
        return carry

    lax.fori_loop(0, CVT_ROWS // CVT_CHUNK, chunk, 0)

    @pl.when(wid == 0)
    def _():
        tail = NW * CVT_ROWS
        pltpu.sync_copy(table_h.at[pl.ds(tail, 4)], inb.at[pl.ds(0, 4)])
        convert_rows(4)
        pltpu.sync_copy(outb.at[pl.ds(0, 4)], out_h.at[pl.ds(tail, 4)])


@functools.partial(
    pl.kernel,
    out_type=jax.ShapeDtypeStruct((B, D), jnp.float32),
    mesh=_MESH,
    compiler_params=_SC_PARAMS,
    scratch_types=[
        pltpu.VMEM((RPW, LP), jnp.int32),
        pltpu.VMEM((L, D), jnp.bfloat16),
        pltpu.VMEM((L, D), jnp.bfloat16),
        pltpu.VMEM((RPW, D), jnp.float32),
        pltpu.SemaphoreType.DMA,
        pltpu.SemaphoreType.DMA,
    ],
)
def _avg_embed(code_h, table_h, out_h, idx_v, buf0, buf1, out_v, sem0, sem1):
    wid = lax.axis_index("s") * NC + lax.axis_index("c")
    base = wid * RPW

    pltpu.sync_copy(code_h.at[pl.ds(base, RPW)], idx_v.at[:, pl.ds(0, L)])

    bufs = (buf0, buf1)
    sems = (sem0, sem1)

    def start(r, b):
        pltpu.async_copy(
            table_h.at[idx_v.at[r, pl.ds(0, SPLIT)]],
            bufs[b].at[pl.ds(0, SPLIT)],
            sems[b],
        )
        pltpu.async_copy(
            table_h.at[idx_v.at[r, pl.ds(SPLIT, L - SPLIT)]],
            bufs[b].at[pl.ds(SPLIT, L - SPLIT)],
            sems[b],
        )

    def wait(b):
        pltpu.make_async_copy(table_h.at[pl.ds(0, L)], bufs[b], sems[b]).wait()

    lane = lax.iota(jnp.int32, 16)
    himask = jnp.full((16,), -65536, jnp.int32)

    def reduce_row(buf, r):
        def cbody(k, cv):
            v = idx_v[r, pl.ds(k * 16, 16)]
            return cv + jnp.where(v != 0, 1.0, 0.0).astype(jnp.float32)

        cv = lax.fori_loop(0, 12, cbody, jnp.zeros((16,), jnp.float32),
                           unroll=4)
        vtail = idx_v[r, pl.ds(192, 16)]
        cv = cv + jnp.where((vtail != 0) & (lane < 8), 1.0, 0.0).astype(
            jnp.float32)
        cnt = jnp.broadcast_to(jnp.sum(cv), (16,))

        def sbody(l, accs):
            a0, a1, a2, a3 = accs
            w0 = plsc.bitcast(buf[l, pl.ds(0, 32)], jnp.int32)
            w1 = plsc.bitcast(buf[l, pl.ds(32, 32)], jnp.int32)
            return (
                a0 + plsc.bitcast(lax.shift_left(w0, 16), jnp.float32),
                a1 + plsc.bitcast(w0 & himask, jnp.float32),
                a2 + plsc.bitcast(lax.shift_left(w1, 16), jnp.float32),
                a3 + plsc.bitcast(w1 & himask, jnp.float32),
            )

        z = jnp.zeros((16,), jnp.float32)
        a0, a1, a2, a3 = lax.fori_loop(0, L, sbody, (z, z, z, z), unroll=8)
        out_v[r, pl.ds(0, 16)] = a0 / cnt
        out_v[r, pl.ds(16, 16)] = a1 / cnt
        out_v[r, pl.ds(32, 16)] = a2 / cnt
        out_v[r, pl.ds(48, 16)] = a3 / cnt

    start(0, 0)

    def gbody(g, carry):
        r0 = 2 * g
        start(r0 + 1, 1)
        wait(0)
        reduce_row(buf0, r0)

        @pl.when(g < RPW // 2 - 1)
        def _():
            start(r0 + 2, 0)

        wait(1)
        reduce_row(buf1, r0 + 1)
        return carry

    lax.fori_loop(0, RPW // 2, gbody, 0)

    pltpu.sync_copy(out_v, out_h.at[pl.ds(base, RPW)])


def kernel(code, code_table):
    return _avg_embed(code.astype(jnp.int32), _to_bf16(code_table))


# speedup vs baseline: 1.1292x; 1.1292x over previous
import functools

import jax
import jax.numpy as jnp
import numpy as np
from jax import lax
from jax.experimental import pallas as pl
from jax.experimental.pallas import tpu as pltpu
from jax.experimental.pallas import tpu_sc as plsc

B = 4096
L = 200
D = 64
V = 100004
NC = 2
NS = 16
NW = NC * NS
RPW = B // NW
LP = 208
SPLIT = 128

_DEINT = [c for c in range(0, 32, 2)] + [c for c in range(1, 32, 2)]
_PERM = _DEINT + [32 + c for c in _DEINT]
_INV_PERM = np.argsort(np.array(_PERM))


@functools.partial(
    pl.kernel,
    out_type=jax.ShapeDtypeStruct((B, D), jnp.float32),
    mesh=plsc.VectorSubcoreMesh(core_axis_name="c", subcore_axis_name="s"),
    compiler_params=pltpu.CompilerParams(
        use_tc_tiling_on_sc=False, needs_layout_passes=False),
    scratch_types=[
        pltpu.VMEM((RPW, LP), jnp.int32),
        pltpu.VMEM((L, D), jnp.bfloat16),
        pltpu.VMEM((L, D), jnp.bfloat16),
        pltpu.VMEM((RPW, D), jnp.float32),
        pltpu.SemaphoreType.DMA,
        pltpu.SemaphoreType.DMA,
    ],
)
def _avg_embed(code_h, table_h, out_h, idx_v, buf0, buf1, out_v, sem0, sem1):
    wid = lax.axis_index("s") * NC + lax.axis_index("c")
    base = wid * RPW

    pltpu.sync_copy(code_h.at[pl.ds(base, RPW)], idx_v.at[:, pl.ds(0, L)])

    bufs = (buf0, buf1)
    sems = (sem0, sem1)

    def start(r, b):
        pltpu.async_copy(
            table_h.at[idx_v.at[r, pl.ds(0, SPLIT)]],
            bufs[b].at[pl.ds(0, SPLIT)],
            sems[b],
        )
        pltpu.async_copy(
            table_h.at[idx_v.at[r, pl.ds(SPLIT, L - SPLIT)]],
            bufs[b].at[pl.ds(SPLIT, L - SPLIT)],
            sems[b],
        )

    def wait(b):
        pltpu.make_async_copy(table_h.at[pl.ds(0, L)], bufs[b], sems[b]).wait()

    lane = lax.iota(jnp.int32, 16)

    def reduce_row(buf, r):
        def cbody(k, cv):
            v = idx_v[r, pl.ds(k * 16, 16)]
            return cv + jnp.where(v != 0, 1.0, 0.0).astype(jnp.float32)

        cv = lax.fori_loop(0, 12, cbody, jnp.zeros((16,), jnp.float32),
                           unroll=4)
        vtail = idx_v[r, pl.ds(192, 16)]
        cv = cv + jnp.where((vtail != 0) & (lane < 8), 1.0, 0.0).astype(
            jnp.float32)
        cnt = jnp.broadcast_to(jnp.sum(cv), (16,))

        def sbody(l, accs):
            a0, a1, a2, a3 = accs
            e0, o0 = plsc.unpack(buf[l, pl.ds(0, 32)],
                                 format=plsc.PackFormat.INTERLEAVED)
            e1, o1 = plsc.unpack(buf[l, pl.ds(32, 32)],
                                 format=plsc.PackFormat.INTERLEAVED)
            return (a0 + e0, a1 + o0, a2 + e1, a3 + o1)

        z = jnp.zeros((16,), jnp.float32)
        a0, a1, a2, a3 = lax.fori_loop(0, L, sbody, (z, z, z, z), unroll=8)
        out_v[r, pl.ds(0, 16)] = a0 / cnt
        out_v[r, pl.ds(16, 16)] = a1 / cnt
        out_v[r, pl.ds(32, 16)] = a2 / cnt
        out_v[r, pl.ds(48, 16)] = a3 / cnt

    start(0, 0)

    def gbody(g, carry):
        r0 = 2 * g
        start(r0 + 1, 1)
        wait(0)
        reduce_row(buf0, r0)

        @pl.when(g < RPW // 2 - 1)
        def _():
            start(r0 + 2, 0)

        wait(1)
        reduce_row(buf1, r0 + 1)
        return carry

    lax.fori_loop(0, RPW // 2, gbody, 0)

    pltpu.sync_copy(out_v, out_h.at[pl.ds(base, RPW)])


def kernel(code, code_table):
    table_bf16 = code_table.reshape(-1).astype(jnp.bfloat16).reshape(V, D)
    out_raw = _avg_embed(code.astype(jnp.int32), table_bf16)
    return out_raw[:, _INV_PERM]
